# baseline (speedup 1.0000x reference)
import functools

import jax
import jax.numpy as jnp
from jax import lax
from jax.experimental import pallas as pl
from jax.experimental.pallas import tpu as pltpu
from jax.experimental.pallas import tpu_sc as plsc

N_FIELDS = 26
BATCH = 16384
VOCAB = 100000
EMBED = 32
N_NUM = 13
OUT_DIM = N_FIELDS * EMBED + N_NUM
FEAT_PAD = 896
N_GROUPS = 7
EPS = 1e-5

NC, NS = 2, 16
NW = NC * NS
CB = BATCH // NW


def _sc_gather(categorical, numerical_pad, tables_flat):
    mesh = plsc.VectorSubcoreMesh(core_axis_name="c", subcore_axis_name="s")

    @functools.partial(
        pl.kernel,
        mesh=mesh,
        out_type=(
            jax.ShapeDtypeStruct((N_GROUPS, BATCH, 128), jnp.float32),
            jax.ShapeDtypeStruct((NW, 2 * N_GROUPS, 128), jnp.float32),
        ),
        compiler_params=pltpu.CompilerParams(use_tc_tiling_on_sc=False),
        scratch_types=[
            pltpu.VMEM((N_FIELDS, CB), jnp.int32),
            pltpu.VMEM((CB, 128), jnp.float32),
            pltpu.VMEM((2 * N_GROUPS, 128), jnp.float32),
            pltpu.SemaphoreType.DMA,
            pltpu.SemaphoreType.DMA,
        ],
    )
    def body(cat_hbm, num_hbm, tab_hbm, feat_hbm, stat_hbm,
             idx_v, out_v, stat_v, gsem, dsem):
        wid = lax.axis_index("s") * NC + lax.axis_index("c")
        base = pl.multiple_of(wid * CB, CB)
        pltpu.sync_copy(cat_hbm.at[:, pl.ds(base, CB)], idx_v)

        def group_step(g, carry):
            for j in range(4):
                f = g * 4 + j

                @pl.when(f < N_FIELDS)
                def _field():
                    foff = f * VOCAB

                    def row_copy16(t, c):
                        vec = idx_v[f, pl.ds(t * 16, 16)]
                        for k in range(16):
                            gidx = vec[k] + foff
                            pltpu.async_copy(
                                tab_hbm.at[gidx, :],
                                out_v.at[t * 16 + k, pl.ds(j * EMBED, EMBED)],
                                gsem,
                            )
                        return c

                    lax.fori_loop(0, CB // 16, row_copy16, 0)

            @pl.when(g < N_GROUPS - 1)
            def _drain_full():
                pltpu.make_async_copy(
                    feat_hbm.at[g, pl.ds(base, CB), :], out_v, gsem
                ).wait()

            @pl.when(g == N_GROUPS - 1)
            def _drain_half():
                pltpu.make_async_copy(
                    feat_hbm.at[g, pl.ds(base, CB // 2), :],
                    out_v.at[pl.ds(0, CB // 2), :],
                    gsem,
                ).wait()

            @pl.when(g == N_GROUPS - 1)
            def _numerical():
                def num_copy16(r16, c):
                    for k in range(16):
                        r = r16 * 16 + k
                        pltpu.async_copy(
                            num_hbm.at[base + r, :],
                            out_v.at[r, pl.ds(64, 16)],
                            gsem,
                        )
                    return c

                lax.fori_loop(0, CB // 16, num_copy16, 0)
                pltpu.make_async_copy(
                    feat_hbm.at[g, pl.ds(base, 64), :],
                    out_v.at[pl.ds(0, 64), :],
                    gsem,
                ).wait()

            pltpu.async_copy(
                out_v, feat_hbm.at[g, pl.ds(base, CB), :], dsem
            )

            def stat_row(r, acc):
                sums, sqs = list(acc[:8]), list(acc[8:])
                for h in range(8):
                    v = out_v[r, pl.ds(16 * h, 16)]
                    sums[h] = sums[h] + v
                    sqs[h] = sqs[h] + v * v
                return tuple(sums) + tuple(sqs)

            zero = jnp.zeros((16,), jnp.float32)
            acc = lax.fori_loop(0, CB, stat_row, (zero,) * 16)
            for h in range(8):
                stat_v[2 * g, pl.ds(16 * h, 16)] = acc[h]
                stat_v[2 * g + 1, pl.ds(16 * h, 16)] = acc[8 + h]

            pltpu.make_async_copy(
                out_v, feat_hbm.at[g, pl.ds(base, CB), :], dsem
            ).wait()
            return carry

        lax.fori_loop(0, N_GROUPS, group_step, 0)
        pltpu.sync_copy(stat_v, stat_hbm.at[wid])

    return body(categorical, numerical_pad, tables_flat)


def _tc_batchnorm(features, stats, gamma_g, beta_g):
    BB = 2048
    nb = BATCH // BB

    def body(feat_ref, stat_ref, g_ref, b_ref, out_ref, scale_ref, bias_ref):
        g = pl.program_id(0)
        blk = pl.program_id(1)

        @pl.when(blk == 0)
        def _reduce_stats():
            inv_n = jnp.float32(1.0 / BATCH)
            s = stat_ref[:, pl.ds(2 * g, 1), :][:, 0, :]
            q = stat_ref[:, pl.ds(2 * g + 1, 1), :][:, 0, :]
            mean = jnp.sum(s, axis=0, keepdims=True) * inv_n
            var = jnp.maximum(
                jnp.sum(q, axis=0, keepdims=True) * inv_n - mean * mean, 0.0)
            gamma_row = g_ref[pl.ds(g, 1), :]
            beta_row = b_ref[pl.ds(g, 1), :]
            scale_ref[...] = gamma_row * lax.rsqrt(var + EPS)
            bias_ref[...] = beta_row - mean * scale_ref[...]

        out_ref[...] = feat_ref[0] * scale_ref[...] + bias_ref[...]

    return pl.pallas_call(
        body,
        grid=(N_GROUPS, nb),
        in_specs=[
            pl.BlockSpec((1, BB, 128), lambda g, b: (g, b, 0)),
            pl.BlockSpec((NW, 2 * N_GROUPS, 128), lambda g, b: (0, 0, 0)),
            pl.BlockSpec((N_GROUPS, 128), lambda g, b: (0, 0)),
            pl.BlockSpec((N_GROUPS, 128), lambda g, b: (0, 0)),
        ],
        out_specs=pl.BlockSpec((BB, 128), lambda g, b: (b, g)),
        out_shape=jax.ShapeDtypeStruct((BATCH, OUT_DIM), jnp.float32),
        scratch_shapes=[
            pltpu.VMEM((1, 128), jnp.float32),
            pltpu.VMEM((1, 128), jnp.float32),
        ],
    )(features, stats, gamma_g, beta_g)


def kernel(categorical, numerical, tables, gamma, beta):
    numerical_pad = jnp.pad(numerical.astype(jnp.float32), ((0, 0), (0, 16 - N_NUM)))
    tables_flat = tables.astype(jnp.float32).reshape(N_FIELDS * VOCAB, EMBED)
    features, stats = _sc_gather(categorical.astype(jnp.int32), numerical_pad,
                                 tables_flat)

    gamma_g = jnp.pad(gamma.astype(jnp.float32), (0, FEAT_PAD - OUT_DIM),
                      constant_values=1.0).reshape(N_GROUPS, 128)
    beta_g = jnp.pad(beta.astype(jnp.float32), (0, FEAT_PAD - OUT_DIM)).reshape(N_GROUPS, 128)
    return _tc_batchnorm(features, stats, gamma_g, beta_g)


# speedup vs baseline: 2.6540x; 2.6540x over previous
import functools

import jax
import jax.numpy as jnp
from jax import lax
from jax.experimental import pallas as pl
from jax.experimental.pallas import tpu as pltpu
from jax.experimental.pallas import tpu_sc as plsc

N_FIELDS = 26
BATCH = 16384
VOCAB = 100000
EMBED = 32
N_NUM = 13
OUT_DIM = N_FIELDS * EMBED + N_NUM
FEAT_PAD = 896
N_GROUPS = 7
EPS = 1e-5

NC, NS = 2, 16
NW = NC * NS
CB = BATCH // NW


def _sc_gather(categorical, numerical_pad, tables_flat):
    mesh = plsc.VectorSubcoreMesh(core_axis_name="c", subcore_axis_name="s")

    @functools.partial(
        pl.kernel,
        mesh=mesh,
        out_type=(
            jax.ShapeDtypeStruct((N_GROUPS, BATCH, 128), jnp.float32),
            jax.ShapeDtypeStruct((NW, 2 * N_GROUPS, 128), jnp.float32),
        ),
        scratch_types=[
            pltpu.VMEM((N_FIELDS, CB), jnp.int32),
            pltpu.VMEM((CB, 128), jnp.float32),
            pltpu.VMEM((2 * N_GROUPS, 128), jnp.float32),
            pltpu.SemaphoreType.DMA,
            pltpu.SemaphoreType.DMA,
        ],
    )
    def body(cat_hbm, num_hbm, tab_hbm, feat_hbm, stat_hbm,
             idx_v, out_v, stat_v, gsem, dsem):
        wid = lax.axis_index("s") * NC + lax.axis_index("c")
        base = pl.multiple_of(wid * CB, CB)
        pltpu.sync_copy(cat_hbm.at[:, pl.ds(base, CB)], idx_v)

        def group_step(g, carry):
            for j in range(4):
                f = g * 4 + j

                @pl.when(f < N_FIELDS)
                def _field():
                    foff = f * VOCAB

                    def row_copy16(t, c):
                        vec = idx_v[f, pl.ds(t * 16, 16)]
                        for k in range(16):
                            gidx = vec[k] + foff
                            pltpu.async_copy(
                                tab_hbm.at[gidx, :],
                                out_v.at[t * 16 + k, pl.ds(j * EMBED, EMBED)],
                                gsem,
                            )
                        return c

                    lax.fori_loop(0, CB // 16, row_copy16, 0)

            @pl.when(g < N_GROUPS - 1)
            def _drain_full():
                pltpu.make_async_copy(
                    feat_hbm.at[g, pl.ds(base, CB), :], out_v, gsem
                ).wait()

            @pl.when(g == N_GROUPS - 1)
            def _drain_half():
                pltpu.make_async_copy(
                    feat_hbm.at[g, pl.ds(base, CB // 2), :],
                    out_v.at[pl.ds(0, CB // 2), :],
                    gsem,
                ).wait()

            @pl.when(g == N_GROUPS - 1)
            def _numerical():
                def num_copy16(r16, c):
                    for k in range(16):
                        r = r16 * 16 + k
                        pltpu.async_copy(
                            num_hbm.at[base + r, :],
                            out_v.at[r, pl.ds(64, 16)],
                            gsem,
                        )
                    return c

                lax.fori_loop(0, CB // 16, num_copy16, 0)
                pltpu.make_async_copy(
                    feat_hbm.at[g, pl.ds(base, 64), :],
                    out_v.at[pl.ds(0, 64), :],
                    gsem,
                ).wait()

            pltpu.async_copy(
                out_v, feat_hbm.at[g, pl.ds(base, CB), :], dsem
            )

            def stat_row(r, acc):
                sums, sqs = list(acc[:8]), list(acc[8:])
                for h in range(8):
                    v = out_v[r, pl.ds(16 * h, 16)]
                    sums[h] = sums[h] + v
                    sqs[h] = sqs[h] + v * v
                return tuple(sums) + tuple(sqs)

            zero = jnp.zeros((16,), jnp.float32)
            acc = lax.fori_loop(0, CB, stat_row, (zero,) * 16)
            for h in range(8):
                stat_v[2 * g, pl.ds(16 * h, 16)] = acc[h]
                stat_v[2 * g + 1, pl.ds(16 * h, 16)] = acc[8 + h]

            pltpu.make_async_copy(
                out_v, feat_hbm.at[g, pl.ds(base, CB), :], dsem
            ).wait()
            return carry

        lax.fori_loop(0, N_GROUPS, group_step, 0)
        pltpu.sync_copy(stat_v, stat_hbm.at[wid])

    return body(categorical, numerical_pad, tables_flat)


def _tc_batchnorm(features, stats, gamma_g, beta_g):
    BB = 4096
    nb = BATCH // BB

    def body(feat_ref, stat_ref, g_ref, b_ref, out_ref, scale_ref, bias_ref):
        g = pl.program_id(0)
        blk = pl.program_id(1)

        @pl.when(blk == 0)
        def _reduce_stats():
            inv_n = jnp.float32(1.0 / BATCH)
            s = stat_ref[:, pl.ds(2 * g, 1), :][:, 0, :]
            q = stat_ref[:, pl.ds(2 * g + 1, 1), :][:, 0, :]
            mean = jnp.sum(s, axis=0, keepdims=True) * inv_n
            var = jnp.maximum(
                jnp.sum(q, axis=0, keepdims=True) * inv_n - mean * mean, 0.0)
            gamma_row = g_ref[pl.ds(g, 1), :]
            beta_row = b_ref[pl.ds(g, 1), :]
            scale_ref[...] = gamma_row * lax.rsqrt(var + EPS)
            bias_ref[...] = beta_row - mean * scale_ref[...]

        out_ref[...] = feat_ref[0] * scale_ref[...] + bias_ref[...]

    return pl.pallas_call(
        body,
        grid=(N_GROUPS, nb),
        in_specs=[
            pl.BlockSpec((1, BB, 128), lambda g, b: (g, b, 0)),
            pl.BlockSpec((NW, 2 * N_GROUPS, 128), lambda g, b: (0, 0, 0)),
            pl.BlockSpec((N_GROUPS, 128), lambda g, b: (0, 0)),
            pl.BlockSpec((N_GROUPS, 128), lambda g, b: (0, 0)),
        ],
        out_specs=pl.BlockSpec((BB, 128), lambda g, b: (b, g)),
        out_shape=jax.ShapeDtypeStruct((BATCH, OUT_DIM), jnp.float32),
        scratch_shapes=[
            pltpu.VMEM((1, 128), jnp.float32),
            pltpu.VMEM((1, 128), jnp.float32),
        ],
    )(features, stats, gamma_g, beta_g)


def kernel(categorical, numerical, tables, gamma, beta):
    numerical_pad = jnp.pad(numerical.astype(jnp.float32), ((0, 0), (0, 16 - N_NUM)))
    tables_flat = tables.astype(jnp.float32).reshape(N_FIELDS * VOCAB, EMBED)
    features, stats = _sc_gather(categorical.astype(jnp.int32), numerical_pad,
                                 tables_flat)

    gamma_g = jnp.pad(gamma.astype(jnp.float32), (0, FEAT_PAD - OUT_DIM),
                      constant_values=1.0).reshape(N_GROUPS, 128)
    beta_g = jnp.pad(beta.astype(jnp.float32), (0, FEAT_PAD - OUT_DIM)).reshape(N_GROUPS, 128)
    return _tc_batchnorm(features, stats, gamma_g, beta_g)
